# baseline (speedup 1.0000x reference)
import functools

import jax
import jax.numpy as jnp
from jax import lax
from jax.experimental import pallas as pl
from jax.experimental.pallas import tpu as pltpu
from jax.experimental.pallas import tpu_sc as plsc

_D = 32
_BATCH = 4096
_HIST = 200
_B = _BATCH * _HIST

_info = plsc.get_sparse_core_info()
_NC, _NS = _info.num_cores, _info.num_subcores
_NW = _NC * _NS
_BPW = _BATCH // _NW
_IPW = _BPW * _HIST
_HBLK = 2
_NBLK = _HIST // _HBLK
_ROWS = _HBLK * _BPW

_mesh = plsc.VectorSubcoreMesh(core_axis_name="c", subcore_axis_name="s")


@functools.partial(
    pl.kernel,
    mesh=_mesh,
    out_type=jax.ShapeDtypeStruct((_HIST, 4, _NW, 8, 128), jnp.float32),
    scratch_types=[
        pltpu.VMEM((_IPW,), jnp.int32),
        [pltpu.VMEM((_ROWS,), jnp.int32) for _ in range(2)],
        [pltpu.VMEM((_ROWS, _D), jnp.float32) for _ in range(2)],
        [pltpu.VMEM((_HBLK, 4, 8, 128), jnp.float32) for _ in range(2)],
        [pltpu.SemaphoreType.DMA for _ in range(2)],
        [pltpu.SemaphoreType.DMA for _ in range(2)],
    ],
    compiler_params=pltpu.CompilerParams(
        use_tc_tiling_on_sc=False, needs_layout_passes=False
    ),
)
def _embed(idx_hbm, table_hbm, out_hbm, idx_v, hidx, rows, ptile, gsems, ssems):
    wid = lax.axis_index("s") * _NC + lax.axis_index("c")

    pltpu.sync_copy(idx_hbm.at[pl.ds(wid * _IPW, _IPW)], idx_v)

    iota = lax.iota(jnp.int32, 16)
    base200 = [iota * _HIST + 16 * _HIST * k for k in range(8)]
    rowv = [iota + 16 * lk for lk in range(16)]

    def build_hidx(i, u):
        h0 = i * _HBLK
        for hh in range(_HBLK):
            for k in range(8):
                v = plsc.load_gather(idx_v, [base200[k] + (h0 + hh)])
                hidx[u][pl.ds(hh * _BPW + k * 16, 16)] = v

    def start_gather(u):
        pltpu.async_copy(table_hbm.at[hidx[u]], rows[u], gsems[u])

    def wait_gather(u):
        pltpu.make_async_copy(table_hbm.at[hidx[u]], rows[u], gsems[u]).wait()

    def out_slice(i):
        return out_hbm.at[pl.ds(i * _HBLK, _HBLK), :, wid]

    def start_store(i, u):
        pltpu.async_copy(ptile[u], out_slice(i), ssems[u])

    def wait_store(i, u):
        pltpu.make_async_copy(ptile[u], out_slice(i), ssems[u]).wait()

    def transpose(u):
        for hh in range(_HBLK):
            for lk in range(8):
                rv = rowv[hh * 8 + lk]
                for r in range(4):
                    for s in range(8):
                        cv = iota * 0 + (r * 8 + s)
                        val = plsc.load_gather(rows[u], [rv, cv])
                        ptile[u][hh, r, s, pl.ds(lk * 16, 16)] = val

    build_hidx(0, 0)
    start_gather(0)

    def outer(t, carry):
        for u in range(2):
            i = t * 2 + u

            if u == 0:
                build_hidx(i + 1, 1)
                start_gather(1)
            else:

                @pl.when(t < _NBLK // 2 - 1)
                def _():
                    build_hidx(i + 1, 0)
                    start_gather(0)

            wait_gather(u)

            @pl.when(t >= 1)
            def _():
                wait_store(i - 2, u)

            transpose(u)
            start_store(i, u)
        return carry

    lax.fori_loop(0, _NBLK // 2, outer, 0)

    wait_store(_NBLK - 2, 0)
    wait_store(_NBLK - 1, 1)


def kernel(seq, table):
    flat = seq.reshape(-1)
    p = _embed(flat, table)
    return p.transpose(2, 4, 0, 1, 3).reshape(_BATCH, _HIST, _D)


# speedup vs baseline: 1.3670x; 1.3670x over previous
import functools

import jax
import jax.numpy as jnp
from jax import lax
from jax.experimental import pallas as pl
from jax.experimental.pallas import tpu as pltpu
from jax.experimental.pallas import tpu_sc as plsc

_D = 32
_BATCH = 4096
_HIST = 200
_B = _BATCH * _HIST

_info = plsc.get_sparse_core_info()
_NC, _NS = _info.num_cores, _info.num_subcores
_NW = _NC * _NS
_BPW = _BATCH // _NW
_IPW = _BPW * _HIST
_HBLK = 2
_NBLK = _HIST // _HBLK
_ROWS = _HBLK * _BPW

_mesh = plsc.VectorSubcoreMesh(core_axis_name="c", subcore_axis_name="s")


@functools.partial(
    pl.kernel,
    mesh=_mesh,
    out_type=jax.ShapeDtypeStruct((_HIST, 4, _NW, 8, 128), jnp.float32),
    scratch_types=[
        pltpu.VMEM((_IPW,), jnp.int32),
        [pltpu.VMEM((_ROWS,), jnp.int32) for _ in range(2)],
        [pltpu.VMEM((_ROWS, _D), jnp.float32) for _ in range(2)],
        [pltpu.VMEM((_HBLK, 4, 8, 128), jnp.float32) for _ in range(2)],
        [pltpu.SemaphoreType.DMA for _ in range(2)],
        [pltpu.SemaphoreType.DMA for _ in range(2)],
    ],
    compiler_params=pltpu.CompilerParams(
        use_tc_tiling_on_sc=False,
        needs_layout_passes=False,
        disable_bounds_checks=True,
    ),
)
def _embed(idx_hbm, table_hbm, out_hbm, idx_v, hidx, rows, ptile, gsems, ssems):
    wid = lax.axis_index("s") * _NC + lax.axis_index("c")

    pltpu.sync_copy(idx_hbm.at[pl.ds(wid * _IPW, _IPW)], idx_v)

    iota = lax.iota(jnp.int32, 16)
    base200 = [iota * _HIST + 16 * _HIST * k for k in range(8)]
    rowv = [iota + 16 * lk for lk in range(16)]

    def build_hidx(i, u):
        h0 = i * _HBLK
        for hh in range(_HBLK):
            for k in range(8):
                v = plsc.load_gather(idx_v, [base200[k] + (h0 + hh)])
                hidx[u][pl.ds(hh * _BPW + k * 16, 16)] = v

    def start_gather(u):
        pltpu.async_copy(table_hbm.at[hidx[u]], rows[u], gsems[u])

    def wait_gather(u):
        pltpu.make_async_copy(table_hbm.at[hidx[u]], rows[u], gsems[u]).wait()

    def out_slice(i):
        return out_hbm.at[pl.ds(i * _HBLK, _HBLK), :, wid]

    def start_store(i, u):
        pltpu.async_copy(ptile[u], out_slice(i), ssems[u])

    def wait_store(i, u):
        pltpu.make_async_copy(ptile[u], out_slice(i), ssems[u]).wait()

    zero16 = iota * 0

    def transpose(u):
        @plsc.parallel_loop(0, _HBLK * 8 * _D, unroll=8)
        def _(g):
            hh = g >> 8
            lk = (g >> 5) & 7
            d = g & 31
            rv = iota + ((hh << 7) + (lk << 4))
            cv = zero16 + d
            val = plsc.load_gather(rows[u], [rv, cv])
            ptile[u][hh, d >> 3, d & 7, pl.ds(lk * 16, 16)] = val

    build_hidx(0, 0)
    start_gather(0)

    def outer(t, carry):
        for u in range(2):
            i = t * 2 + u

            if u == 0:
                build_hidx(i + 1, 1)
                start_gather(1)
            else:

                @pl.when(t < _NBLK // 2 - 1)
                def _():
                    build_hidx(i + 1, 0)
                    start_gather(0)

            wait_gather(u)

            @pl.when(t >= 1)
            def _():
                wait_store(i - 2, u)

            transpose(u)
            start_store(i, u)
        return carry

    lax.fori_loop(0, _NBLK // 2, outer, 0)

    wait_store(_NBLK - 2, 0)
    wait_store(_NBLK - 1, 1)


def kernel(seq, table):
    flat = seq.reshape(-1)
    p = _embed(flat, table)
    return p.transpose(2, 4, 0, 1, 3).reshape(_BATCH, _HIST, _D)
